# baseline (speedup 1.0000x reference)
import functools
import math

import numpy as np
import jax
import jax.numpy as jnp
from jax import lax
from jax.experimental import pallas as pl
from jax.experimental.pallas import tpu as pltpu
from jax.experimental.pallas import tpu_sc as plsc

T = 6
C = 128
G = 1024
HEADS = 4
HEAD_DIM = C // HEADS
K_KEEP = 4
IH = T * HEADS
PACK = T * IH
BN = 256


def _build_consts():
    r62 = np.zeros((2 * T * C, IH), np.float32)
    for half in range(2):
        for i in range(T):
            for d in range(C):
                r62[half * T * C + i * C + d, i * HEADS + d // HEAD_DIM] = 1.0
    jidx2 = np.zeros((PACK,), np.int32)
    for j in range(T):
        for q in range(IH):
            jidx2[j * IH + q] = j
    pout2 = np.zeros((PACK, PACK), np.float32)
    for j in range(T):
        for i in range(T):
            for h in range(HEADS):
                pout2[j * IH + i * HEADS + h, h * T * T + i * T + j] = 1.0
    bbig24 = np.zeros((IH, T * C), np.float32)
    for i in range(T):
        for h in range(HEADS):
            for d in range(C):
                if d // HEAD_DIM == h:
                    bbig24[i * HEADS + h, i * C + d] = 1.0
    return r62, jidx2, pout2, bbig24


_R62, _JIDX2, _POUT2, _BBIG24 = _build_consts()


def _attn_kernel(emb_ref, acg_ref, wkt_ref, bk_ref, wqt_ref, bq_ref,
                 wgj_ref, bgj_ref, r62_ref, pout_ref, bbig_ref,
                 ctx_ref, attn_ref):
    f32 = jnp.float32
    bf16 = jnp.bfloat16
    xs = [emb_ref[:, i * C:(i + 1) * C] for i in range(T)]
    xbs = [x.astype(bf16) for x in xs]
    wkt = wkt_ref[...]
    wqt = wqt_ref[...]
    bk = bk_ref[...]
    bq = bq_ref[...]
    ks = [jnp.dot(x, wkt, preferred_element_type=f32) + bk for x in xbs]
    qs = [jnp.dot(x, wqt, preferred_element_type=f32) + bq for x in xbs]
    kbs = [k.astype(bf16).astype(f32) for k in ks]
    qbs = [q.astype(bf16).astype(f32) for q in qs]

    gate = jnp.dot(acg_ref[...].astype(bf16), wgj_ref[...],
                   preferred_element_type=f32) + bgj_ref[...]

    rsqrt_hd = f32(math.sqrt(HEAD_DIM))
    r62 = r62_ref[...]
    s = []
    for j in range(T):
        prod = jnp.concatenate([qbs[i] * kbs[j] for i in range(T)], axis=1)
        hi = prod.astype(bf16)
        lo = (prod - hi.astype(f32)).astype(bf16)
        sj = jnp.dot(jnp.concatenate([hi, lo], axis=1), r62,
                     preferred_element_type=f32)
        s.append(sj / rsqrt_hd * gate[:, j * IH:(j + 1) * IH])

    ab = [jnp.abs(x) for x in s]
    ranks = [jnp.full_like(ab[j], float(T - 1 - j)) for j in range(T)]
    for x in range(T):
        for y in range(x + 1, T):
            c = (ab[x] >= ab[y]).astype(f32)
            ranks[y] = ranks[y] + c
            ranks[x] = ranks[x] - c
    keep = [r < float(K_KEEP) for r in ranks]

    neg = f32(-3.0e38)
    masked = [jnp.where(keep[j], s[j], neg) for j in range(T)]
    m = masked[0]
    for j in range(1, T):
        m = jnp.maximum(m, masked[j])
    e = [jnp.where(keep[j], jnp.exp(s[j] - m), 0.0) for j in range(T)]
    z = e[0]
    for j in range(1, T):
        z = z + e[j]
    rz = 1.0 / z
    a = [e[j] * rz for j in range(T)]

    ab16 = [x.astype(bf16) for x in a]
    acat = jnp.concatenate(ab16, axis=1)
    attn_ref[...] = jnp.dot(acat, pout_ref[...], preferred_element_type=f32)

    bbig = bbig_ref[...]
    bc = [jnp.dot(x, bbig, preferred_element_type=f32) for x in ab16]
    for i in range(T):
        acc = bc[0][:, i * C:(i + 1) * C] * xs[0]
        for j in range(1, T):
            acc = acc + bc[j][:, i * C:(i + 1) * C] * xs[j]
        ctx_ref[:, i, :] = acc


def _tc_specs(n):
    grid = (n // BN,)

    def rows2(i):
        return (i, 0)

    def const2(i):
        return (0, 0)

    in_specs = [
        pl.BlockSpec((BN, T * C), rows2),
        pl.BlockSpec((BN, 128), rows2),
        pl.BlockSpec((C, C), const2),
        pl.BlockSpec((1, C), const2),
        pl.BlockSpec((C, C), const2),
        pl.BlockSpec((1, C), const2),
        pl.BlockSpec((128, PACK), const2),
        pl.BlockSpec((1, PACK), const2),
        pl.BlockSpec((2 * T * C, IH), const2),
        pl.BlockSpec((PACK, PACK), const2),
        pl.BlockSpec((IH, T * C), const2),
    ]
    out_specs = [
        pl.BlockSpec((BN, T, C), lambda i: (i, 0, 0)),
        pl.BlockSpec((BN, PACK), rows2),
    ]
    out_shape = [
        jax.ShapeDtypeStruct((n, T, C), jnp.float32),
        jax.ShapeDtypeStruct((n, PACK), jnp.float32),
    ]
    return grid, in_specs, out_specs, out_shape


def _run_attention(embeddings, acg, wkt, bk2, wqt, bq2, wgj, bgj):
    n = embeddings.shape[0]
    grid, in_specs, out_specs, out_shape = _tc_specs(n)
    return pl.pallas_call(
        _attn_kernel,
        grid=grid,
        in_specs=in_specs,
        out_specs=out_specs,
        out_shape=out_shape,
        compiler_params=pltpu.CompilerParams(
            dimension_semantics=("parallel",)),
    )(embeddings, acg, wkt, bk2, wqt, bq2, wgj, bgj,
      _R62, jnp.asarray(_POUT2, jnp.bfloat16), jnp.asarray(_BBIG24, jnp.bfloat16))


def _sc_gather(table, idx):
    n = idx.shape[0]
    info = plsc.get_sparse_core_info()
    nc = info.num_cores
    nw = nc * info.num_subcores
    bpw = n // nw
    nch = 4
    ch = bpw // nch
    mesh = plsc.VectorSubcoreMesh(core_axis_name="c", subcore_axis_name="s")

    @functools.partial(
        pl.kernel, mesh=mesh,
        out_type=jax.ShapeDtypeStruct((n, 128), jnp.float32),
        scratch_types=[
            pltpu.VMEM((bpw,), jnp.int32),
            pltpu.VMEM((ch, 128), jnp.float32),
            pltpu.SemaphoreType.DMA,
        ],
    )
    def gk(table_hbm, idx_hbm, out_hbm, idx_v, rows_v, sem):
        wid = lax.axis_index("s") * nc + lax.axis_index("c")
        base = wid * bpw
        pltpu.sync_copy(idx_hbm.at[pl.ds(base, bpw)], idx_v)
        for k in range(nch):
            pltpu.async_copy(
                table_hbm.at[idx_v.at[pl.ds(k * ch, ch)]], rows_v, sem
            ).wait()
            pltpu.sync_copy(rows_v, out_hbm.at[pl.ds(base + k * ch, ch)])

    return gk(table, idx)


def kernel(embeddings, ac, batch, Wk, bk, Wq, bq, Wg, bg):
    n = embeddings.shape[0]
    acp = jnp.concatenate(
        [ac, jnp.zeros((G, 128 - ac.shape[1]), jnp.float32)],
        axis=1)
    acg = _sc_gather(acp, batch)

    wgt = jnp.concatenate(
        [Wg.T, jnp.zeros((128 - Wg.shape[1], T), jnp.float32)], axis=0)
    wgj = jnp.take(wgt, _JIDX2, axis=1).astype(jnp.bfloat16)
    bgj = jnp.take(bg, _JIDX2)[None, :]

    ctx, attn = _run_attention(
        embeddings.reshape(n, T * C), acg,
        Wk.T.astype(jnp.bfloat16), bk.reshape(1, C),
        Wq.T.astype(jnp.bfloat16), bq.reshape(1, C),
        wgj, bgj)
    return ctx, attn.reshape(n, HEADS, T, T)


# speedup vs baseline: 1.0594x; 1.0594x over previous
import functools
import math

import numpy as np
import jax
import jax.numpy as jnp
from jax import lax
from jax.experimental import pallas as pl
from jax.experimental.pallas import tpu as pltpu
from jax.experimental.pallas import tpu_sc as plsc

T = 6
C = 128
G = 1024
HEADS = 4
HEAD_DIM = C // HEADS
K_KEEP = 4
IH = T * HEADS
PACK = T * IH
BN = 512


def _build_consts():
    r62 = np.zeros((2 * T * C, IH), np.float32)
    for half in range(2):
        for i in range(T):
            for d in range(C):
                r62[half * T * C + i * C + d, i * HEADS + d // HEAD_DIM] = 1.0
    jidx2 = np.zeros((PACK,), np.int32)
    for j in range(T):
        for q in range(IH):
            jidx2[j * IH + q] = j
    pout2 = np.zeros((PACK, PACK), np.float32)
    for j in range(T):
        for i in range(T):
            for h in range(HEADS):
                pout2[j * IH + i * HEADS + h, h * T * T + i * T + j] = 1.0
    bbig24 = np.zeros((IH, T * C), np.float32)
    for i in range(T):
        for h in range(HEADS):
            for d in range(C):
                if d // HEAD_DIM == h:
                    bbig24[i * HEADS + h, i * C + d] = 1.0
    return r62, jidx2, pout2, bbig24


_R62, _JIDX2, _POUT2, _BBIG24 = _build_consts()


def _attn_kernel(emb_ref, acg_ref, wkt_ref, bk_ref, wqt_ref, bq_ref,
                 wgj_ref, bgj_ref, r62_ref, pout_ref, bbig_ref,
                 ctx_ref, attn_ref):
    f32 = jnp.float32
    bf16 = jnp.bfloat16
    xs = [emb_ref[:, i * C:(i + 1) * C] for i in range(T)]
    xbs = [x.astype(bf16) for x in xs]
    wkt = wkt_ref[...]
    wqt = wqt_ref[...]
    bk = bk_ref[...]
    bq = bq_ref[...]
    ks = [jnp.dot(x, wkt, preferred_element_type=f32) + bk for x in xbs]
    qs = [jnp.dot(x, wqt, preferred_element_type=f32) + bq for x in xbs]
    kbs = [k.astype(bf16).astype(f32) for k in ks]
    qbs = [q.astype(bf16).astype(f32) for q in qs]

    gate = jnp.dot(acg_ref[...].astype(bf16), wgj_ref[...],
                   preferred_element_type=f32) + bgj_ref[...]

    inv_rsqrt = f32(1.0 / math.sqrt(HEAD_DIM))
    r62 = r62_ref[...]
    s = []
    for j in range(T):
        prod = jnp.concatenate([qbs[i] * kbs[j] for i in range(T)], axis=1)
        hi = prod.astype(bf16)
        lo = (prod - hi.astype(f32)).astype(bf16)
        sj = jnp.dot(jnp.concatenate([hi, lo], axis=1), r62,
                     preferred_element_type=f32)
        s.append(sj * inv_rsqrt * gate[:, j * IH:(j + 1) * IH])

    ab = [jnp.abs(x) for x in s]
    ranks = [jnp.full_like(ab[j], float(T - 1 - j)) for j in range(T)]
    for x in range(T):
        for y in range(x + 1, T):
            c = (ab[x] >= ab[y]).astype(f32)
            ranks[y] = ranks[y] + c
            ranks[x] = ranks[x] - c
    keep = [r < float(K_KEEP) for r in ranks]

    neg = f32(-3.0e38)
    masked = [jnp.where(keep[j], s[j], neg) for j in range(T)]
    m = masked[0]
    for j in range(1, T):
        m = jnp.maximum(m, masked[j])
    e = [jnp.where(keep[j], jnp.exp(s[j] - m), 0.0) for j in range(T)]
    z = e[0]
    for j in range(1, T):
        z = z + e[j]
    rz = 1.0 / z
    a = [e[j] * rz for j in range(T)]

    ab16 = [x.astype(bf16) for x in a]
    acat = jnp.concatenate(ab16, axis=1)
    attn_ref[...] = jnp.dot(acat, pout_ref[...], preferred_element_type=f32)

    bbig = bbig_ref[...]
    bc = [jnp.dot(x, bbig, preferred_element_type=f32) for x in ab16]
    for i in range(T):
        acc = bc[0][:, i * C:(i + 1) * C] * xs[0]
        for j in range(1, T):
            acc = acc + bc[j][:, i * C:(i + 1) * C] * xs[j]
        ctx_ref[:, i, :] = acc


def _tc_specs(n):
    grid = (n // BN,)

    def rows2(i):
        return (i, 0)

    def const2(i):
        return (0, 0)

    in_specs = [
        pl.BlockSpec((BN, T * C), rows2),
        pl.BlockSpec((BN, 128), rows2),
        pl.BlockSpec((C, C), const2),
        pl.BlockSpec((1, C), const2),
        pl.BlockSpec((C, C), const2),
        pl.BlockSpec((1, C), const2),
        pl.BlockSpec((128, PACK), const2),
        pl.BlockSpec((1, PACK), const2),
        pl.BlockSpec((2 * T * C, IH), const2),
        pl.BlockSpec((PACK, PACK), const2),
        pl.BlockSpec((IH, T * C), const2),
    ]
    out_specs = [
        pl.BlockSpec((BN, T, C), lambda i: (i, 0, 0)),
        pl.BlockSpec((BN, PACK), rows2),
    ]
    out_shape = [
        jax.ShapeDtypeStruct((n, T, C), jnp.float32),
        jax.ShapeDtypeStruct((n, PACK), jnp.float32),
    ]
    return grid, in_specs, out_specs, out_shape


def _run_attention(embeddings, acg, wkt, bk2, wqt, bq2, wgj, bgj):
    n = embeddings.shape[0]
    grid, in_specs, out_specs, out_shape = _tc_specs(n)
    return pl.pallas_call(
        _attn_kernel,
        grid=grid,
        in_specs=in_specs,
        out_specs=out_specs,
        out_shape=out_shape,
        compiler_params=pltpu.CompilerParams(
            dimension_semantics=("parallel",)),
    )(embeddings, acg, wkt, bk2, wqt, bq2, wgj, bgj,
      _R62, jnp.asarray(_POUT2, jnp.bfloat16), jnp.asarray(_BBIG24, jnp.bfloat16))


def _sc_gather(table, idx):
    n = idx.shape[0]
    info = plsc.get_sparse_core_info()
    nc = info.num_cores
    nw = nc * info.num_subcores
    bpw = n // nw
    nch = 4
    ch = bpw // nch
    mesh = plsc.VectorSubcoreMesh(core_axis_name="c", subcore_axis_name="s")

    @functools.partial(
        pl.kernel, mesh=mesh,
        out_type=jax.ShapeDtypeStruct((n, 128), jnp.float32),
        scratch_types=[
            pltpu.VMEM((bpw,), jnp.int32),
            pltpu.VMEM((ch, 128), jnp.float32),
            pltpu.VMEM((ch, 128), jnp.float32),
            pltpu.SemaphoreType.DMA,
            pltpu.SemaphoreType.DMA,
        ],
    )
    def gk(table_hbm, idx_hbm, out_hbm, idx_v, rows_a, rows_b, sem_a, sem_b):
        wid = lax.axis_index("s") * nc + lax.axis_index("c")
        base = wid * bpw
        pltpu.sync_copy(idx_hbm.at[pl.ds(base, bpw)], idx_v)
        bufs = (rows_a, rows_b)
        sems = (sem_a, sem_b)
        cps = []
        for k in range(nch):
            cps.append(pltpu.async_copy(
                table_hbm.at[idx_v.at[pl.ds(k * ch, ch)]],
                bufs[k % 2], sems[k % 2]))
            if k >= 1:
                cps[k - 1].wait()
                pltpu.sync_copy(bufs[(k - 1) % 2],
                                out_hbm.at[pl.ds(base + (k - 1) * ch, ch)])
        cps[nch - 1].wait()
        pltpu.sync_copy(bufs[(nch - 1) % 2],
                        out_hbm.at[pl.ds(base + (nch - 1) * ch, ch)])

    return gk(table, idx)


def kernel(embeddings, ac, batch, Wk, bk, Wq, bq, Wg, bg):
    n = embeddings.shape[0]
    acp = jnp.concatenate(
        [ac, jnp.zeros((G, 128 - ac.shape[1]), jnp.float32)],
        axis=1)
    acg = _sc_gather(acp, batch)

    wgt = jnp.concatenate(
        [Wg.T, jnp.zeros((128 - Wg.shape[1], T), jnp.float32)], axis=0)
    wgj = jnp.take(wgt, _JIDX2, axis=1).astype(jnp.bfloat16)
    bgj = jnp.take(bg, _JIDX2)[None, :]

    ctx, attn = _run_attention(
        embeddings.reshape(n, T * C), acg,
        Wk.T.astype(jnp.bfloat16), bk.reshape(1, C),
        Wq.T.astype(jnp.bfloat16), bq.reshape(1, C),
        wgj, bgj)
    return ctx, attn.reshape(n, HEADS, T, T)
